# baseline (speedup 1.0000x reference)
import functools

import jax
import jax.numpy as jnp
import numpy as np
from jax import lax
from jax.experimental import pallas as pl
from jax.experimental.pallas import tpu as pltpu
from jax.experimental.pallas import tpu_sc as plsc

N = 10000
D = 128
E = 320000
NC = 2
NS = 16
NW = NC * NS
B = 128
CH = 80
EPW = CH * B
EP = NW * EPW
PADE = EP - E
NR = 10240
STRIPE = NR // NS

_mesh = plsc.VectorSubcoreMesh(core_axis_name="c", subcore_axis_name="s")


def _sc_degree(dstp, zrows, onesrows):
    @functools.partial(
        pl.kernel,
        out_type=jax.ShapeDtypeStruct((NC, NR, D), jnp.float32),
        mesh=_mesh,
        scratch_types=[
            pltpu.VMEM((CH, B), jnp.int32),
            pltpu.VMEM((B, D), jnp.float32),
            pltpu.VMEM_SHARED((NR, D), jnp.float32),
        ],
    )
    def k(dstp_hbm, zeros_hbm, ones_hbm, out_hbm, idx_v, ones_v, degw):
        cid = lax.axis_index("c")
        sid = lax.axis_index("s")
        wid = cid * NS + sid
        row0 = sid * STRIPE
        pltpu.sync_copy(zeros_hbm.at[pl.ds(row0, STRIPE)],
                        degw.at[pl.ds(row0, STRIPE)])
        pltpu.sync_copy(dstp_hbm.at[wid], idx_v)
        pltpu.sync_copy(ones_hbm, ones_v)
        plsc.subcore_barrier()

        def body(ch, carry):
            pltpu.sync_copy(ones_v, degw.at[idx_v.at[ch]], add=True)
            return carry

        lax.fori_loop(0, CH, body, 0)
        plsc.subcore_barrier()
        pltpu.sync_copy(degw.at[pl.ds(row0, STRIPE)],
                        out_hbm.at[cid, pl.ds(row0, STRIPE)])

    return k(dstp, zrows, onesrows)


def _sc_scatter(y, srcp, dstp, zrows):
    @functools.partial(
        pl.kernel,
        out_type=jax.ShapeDtypeStruct((NC, NR, D), jnp.float32),
        mesh=_mesh,
        scratch_types=[
            pltpu.VMEM((CH // 2, B), jnp.int32),
            pltpu.VMEM((CH // 2, B), jnp.int32),
            pltpu.VMEM((B, D), jnp.float32),
            pltpu.VMEM((B, D), jnp.float32),
            pltpu.VMEM_SHARED((NR, D), jnp.float32),
            pltpu.SemaphoreType.DMA,
            pltpu.SemaphoreType.DMA,
        ],
    )
    def k(y_hbm, srcp_hbm, dstp_hbm, z_hbm, out_hbm,
          sidx, didx, rows_a, rows_b, acc, sem_a, sem_b):
        cid = lax.axis_index("c")
        sid = lax.axis_index("s")
        wid = cid * NS + sid
        row0 = sid * STRIPE
        CH2 = CH // 2
        pltpu.sync_copy(z_hbm.at[pl.ds(row0, STRIPE)],
                        acc.at[pl.ds(row0, STRIPE)])
        plsc.subcore_barrier()

        for h in range(2):
            pltpu.sync_copy(srcp_hbm.at[wid, pl.ds(h * CH2, CH2)], sidx)
            pltpu.sync_copy(dstp_hbm.at[wid, pl.ds(h * CH2, CH2)], didx)
            pltpu.async_copy(y_hbm.at[sidx.at[0]], rows_a, sem_a)

            def body(g, carry):
                ca = 2 * g
                pltpu.make_async_copy(y_hbm.at[sidx.at[ca]], rows_a,
                                      sem_a).wait()
                pltpu.async_copy(y_hbm.at[sidx.at[ca + 1]], rows_b, sem_b)
                pltpu.sync_copy(rows_a, acc.at[didx.at[ca]], add=True)
                pltpu.make_async_copy(y_hbm.at[sidx.at[ca + 1]], rows_b,
                                      sem_b).wait()
                nxt = jnp.minimum(ca + 2, CH2 - 1)
                pltpu.async_copy(y_hbm.at[sidx.at[nxt]], rows_a, sem_a)
                pltpu.sync_copy(rows_b, acc.at[didx.at[ca + 1]], add=True)
                return carry

            lax.fori_loop(0, CH2 // 2, body, 0)
            pltpu.make_async_copy(y_hbm.at[sidx.at[CH2 - 1]], rows_a,
                                  sem_a).wait()
        plsc.subcore_barrier()
        pltpu.sync_copy(acc.at[pl.ds(row0, STRIPE)],
                        out_hbm.at[cid, pl.ds(row0, STRIPE)])

    return k(y, srcp, dstp, zrows)


_R = 2000


def _tc_matmul(x, W1):
    def body(x_ref, w_ref, t_ref):
        t_ref[...] = jnp.dot(x_ref[...], w_ref[...],
                             preferred_element_type=jnp.float32)

    return pl.pallas_call(
        body,
        grid=(N // _R,),
        in_specs=[
            pl.BlockSpec((_R, D), lambda i: (i, 0)),
            pl.BlockSpec((D, D), lambda i: (0, 0)),
        ],
        out_specs=pl.BlockSpec((_R, D), lambda i: (i, 0)),
        out_shape=jax.ShapeDtypeStruct((N, D), jnp.float32),
    )(x, W1)


def _tc_scale(t1, d0, d1):
    def body(t_ref, d0_ref, d1_ref, y_ref, dinv_ref):
        dinv = lax.rsqrt(1.0 + d0_ref[...] + d1_ref[...])
        y_ref[...] = t_ref[...] * dinv
        dinv_ref[...] = dinv

    return pl.pallas_call(
        body,
        grid=(N // _R,),
        in_specs=[
            pl.BlockSpec((_R, D), lambda i: (i, 0)),
            pl.BlockSpec((_R, 1), lambda i: (i, 0)),
            pl.BlockSpec((_R, 1), lambda i: (i, 0)),
        ],
        out_specs=[
            pl.BlockSpec((_R, D), lambda i: (i, 0)),
            pl.BlockSpec((_R, 1), lambda i: (i, 0)),
        ],
        out_shape=[
            jax.ShapeDtypeStruct((N, D), jnp.float32),
            jax.ShapeDtypeStruct((N, 1), jnp.float32),
        ],
    )(t1, d0, d1)


def _tc_layer2(a0, a1, y1, dinv, b1, W2):
    def body(a0_ref, a1_ref, y1_ref, dinv_ref, b_ref, w_ref, y2_ref):
        dinv = dinv_ref[...]
        h = jnp.maximum(
            dinv * (a0_ref[...] + a1_ref[...] + y1_ref[...]) + b_ref[...],
            0.0)
        y2_ref[...] = jnp.dot(h, w_ref[...],
                              preferred_element_type=jnp.float32) * dinv

    return pl.pallas_call(
        body,
        grid=(N // _R,),
        in_specs=[
            pl.BlockSpec((_R, D), lambda i: (i, 0)),
            pl.BlockSpec((_R, D), lambda i: (i, 0)),
            pl.BlockSpec((_R, D), lambda i: (i, 0)),
            pl.BlockSpec((_R, 1), lambda i: (i, 0)),
            pl.BlockSpec((1, D), lambda i: (0, 0)),
            pl.BlockSpec((D, D), lambda i: (0, 0)),
        ],
        out_specs=pl.BlockSpec((_R, D), lambda i: (i, 0)),
        out_shape=jax.ShapeDtypeStruct((N, D), jnp.float32),
    )(a0, a1, y1, dinv, b1, W2)


def _tc_final(a0, a1, y2, dinv, b2):
    def body(a0_ref, a1_ref, y2_ref, dinv_ref, b_ref, out_ref):
        out_ref[...] = jnp.maximum(
            dinv_ref[...] * (a0_ref[...] + a1_ref[...] + y2_ref[...])
            + b_ref[...], 0.0)

    return pl.pallas_call(
        body,
        grid=(N // _R,),
        in_specs=[
            pl.BlockSpec((_R, D), lambda i: (i, 0)),
            pl.BlockSpec((_R, D), lambda i: (i, 0)),
            pl.BlockSpec((_R, D), lambda i: (i, 0)),
            pl.BlockSpec((_R, 1), lambda i: (i, 0)),
            pl.BlockSpec((1, D), lambda i: (0, 0)),
        ],
        out_specs=pl.BlockSpec((_R, D), lambda i: (i, 0)),
        out_shape=jax.ShapeDtypeStruct((N, D), jnp.float32),
    )(a0, a1, y2, dinv, b2)


_PAD_SRC = np.asarray((np.arange(PADE) * 13) % N, np.int32)
_PAD_DST = np.asarray(N + np.arange(PADE) % (NR - N), np.int32)


def kernel(x, edge_index, W1, b1, W2, b2):
    src = edge_index[0].astype(jnp.int32)
    dst = edge_index[1].astype(jnp.int32)

    srcp = jnp.concatenate([src, jnp.asarray(_PAD_SRC)]).reshape(NW, CH, B)
    dstp = jnp.concatenate([dst, jnp.asarray(_PAD_DST)]).reshape(NW, CH, B)

    onesrows = jnp.ones((B, D), jnp.float32)
    zrows = jnp.zeros((NR, D), jnp.float32)

    t1 = _tc_matmul(x, W1)
    degp = _sc_degree(dstp, zrows, onesrows)
    d0 = degp[0, :N, 0:1]
    d1 = degp[1, :N, 0:1]

    y1, dinv = _tc_scale(t1, d0, d1)

    acc1 = _sc_scatter(y1, srcp, dstp, zrows)
    y2 = _tc_layer2(acc1[0, :N], acc1[1, :N], y1, dinv,
                    b1.reshape(1, D), W2)

    acc2 = _sc_scatter(y2, srcp, dstp, zrows)
    return _tc_final(acc2[0, :N], acc2[1, :N], y2, dinv, b2.reshape(1, D))


# speedup vs baseline: 1.1048x; 1.1048x over previous
import functools

import jax
import jax.numpy as jnp
import numpy as np
from jax import lax
from jax.experimental import pallas as pl
from jax.experimental.pallas import tpu as pltpu
from jax.experimental.pallas import tpu_sc as plsc

N = 10000
D = 128
E = 320000
NC = 2
NS = 16
NW = NC * NS
B = 128
CH = 80
EPW = CH * B
EP = NW * EPW
PADE = EP - E
NR = 10240
STRIPE = NR // NS

_mesh = plsc.VectorSubcoreMesh(core_axis_name="c", subcore_axis_name="s")


def _sc_degree(dstp, zrows, onesrows):
    @functools.partial(
        pl.kernel,
        out_type=jax.ShapeDtypeStruct((NC, NR, D), jnp.float32),
        mesh=_mesh,
        scratch_types=[
            pltpu.VMEM((CH, B), jnp.int32),
            pltpu.VMEM((B, D), jnp.float32),
            pltpu.VMEM_SHARED((NR, D), jnp.float32),
        ],
    )
    def k(dstp_hbm, zeros_hbm, ones_hbm, out_hbm, idx_v, ones_v, degw):
        cid = lax.axis_index("c")
        sid = lax.axis_index("s")
        wid = cid * NS + sid
        row0 = sid * STRIPE
        pltpu.sync_copy(zeros_hbm.at[pl.ds(row0, STRIPE)],
                        degw.at[pl.ds(row0, STRIPE)])
        pltpu.sync_copy(dstp_hbm.at[wid], idx_v)
        pltpu.sync_copy(ones_hbm, ones_v)
        plsc.subcore_barrier()

        def body(ch, carry):
            pltpu.sync_copy(ones_v, degw.at[idx_v.at[ch]], add=True)
            return carry

        lax.fori_loop(0, CH, body, 0)
        plsc.subcore_barrier()
        pltpu.sync_copy(degw.at[pl.ds(row0, STRIPE)],
                        out_hbm.at[cid, pl.ds(row0, STRIPE)])

    return k(dstp, zrows, onesrows)


def _sc_scatter(y, srcp, dstp, zrows):
    @functools.partial(
        pl.kernel,
        out_type=jax.ShapeDtypeStruct((NC, NR, D), jnp.float32),
        mesh=_mesh,
        scratch_types=[
            pltpu.VMEM((CH // 2, B), jnp.int32),
            pltpu.VMEM((CH // 2, B), jnp.int32),
            pltpu.VMEM((B, D), jnp.float32),
            pltpu.VMEM((B, D), jnp.float32),
            pltpu.VMEM_SHARED((NR, D), jnp.float32),
            pltpu.SemaphoreType.DMA,
            pltpu.SemaphoreType.DMA,
        ],
    )
    def k(y_hbm, srcp_hbm, dstp_hbm, z_hbm, out_hbm,
          sidx, didx, rows_a, rows_b, acc, sem_a, sem_b):
        cid = lax.axis_index("c")
        sid = lax.axis_index("s")
        wid = cid * NS + sid
        row0 = sid * STRIPE
        CH2 = CH // 2
        pltpu.sync_copy(z_hbm.at[pl.ds(row0, STRIPE)],
                        acc.at[pl.ds(row0, STRIPE)])
        plsc.subcore_barrier()

        for h in range(2):
            pltpu.sync_copy(srcp_hbm.at[wid, pl.ds(h * CH2, CH2)], sidx)
            pltpu.sync_copy(dstp_hbm.at[wid, pl.ds(h * CH2, CH2)], didx)
            pltpu.async_copy(y_hbm.at[sidx.at[0]], rows_a, sem_a)
            pltpu.async_copy(y_hbm.at[sidx.at[1]], rows_b, sem_b)

            def body(g, carry):
                ca = 2 * g
                pltpu.make_async_copy(y_hbm.at[sidx.at[ca]], rows_a,
                                      sem_a).wait()
                pltpu.sync_copy(rows_a, acc.at[didx.at[ca]], add=True)
                nxa = jnp.minimum(ca + 2, CH2 - 1)
                pltpu.async_copy(y_hbm.at[sidx.at[nxa]], rows_a, sem_a)
                pltpu.make_async_copy(y_hbm.at[sidx.at[ca + 1]], rows_b,
                                      sem_b).wait()
                pltpu.sync_copy(rows_b, acc.at[didx.at[ca + 1]], add=True)
                nxb = jnp.minimum(ca + 3, CH2 - 1)
                pltpu.async_copy(y_hbm.at[sidx.at[nxb]], rows_b, sem_b)
                return carry

            lax.fori_loop(0, CH2 // 2, body, 0)
            pltpu.make_async_copy(y_hbm.at[sidx.at[CH2 - 1]], rows_a,
                                  sem_a).wait()
            pltpu.make_async_copy(y_hbm.at[sidx.at[CH2 - 1]], rows_b,
                                  sem_b).wait()
        plsc.subcore_barrier()
        pltpu.sync_copy(acc.at[pl.ds(row0, STRIPE)],
                        out_hbm.at[cid, pl.ds(row0, STRIPE)])

    return k(y, srcp, dstp, zrows)


_R = 2000


def _tc_matmul(x, W1):
    def body(x_ref, w_ref, t_ref):
        t_ref[...] = jnp.dot(x_ref[...], w_ref[...],
                             preferred_element_type=jnp.float32)

    return pl.pallas_call(
        body,
        grid=(N // _R,),
        in_specs=[
            pl.BlockSpec((_R, D), lambda i: (i, 0)),
            pl.BlockSpec((D, D), lambda i: (0, 0)),
        ],
        out_specs=pl.BlockSpec((_R, D), lambda i: (i, 0)),
        out_shape=jax.ShapeDtypeStruct((N, D), jnp.float32),
    )(x, W1)


def _tc_scale(t1, d0, d1):
    def body(t_ref, d0_ref, d1_ref, y_ref, dinv_ref):
        dinv = lax.rsqrt(1.0 + d0_ref[...] + d1_ref[...])
        y_ref[...] = t_ref[...] * dinv
        dinv_ref[...] = dinv

    return pl.pallas_call(
        body,
        grid=(N // _R,),
        in_specs=[
            pl.BlockSpec((_R, D), lambda i: (i, 0)),
            pl.BlockSpec((_R, 1), lambda i: (i, 0)),
            pl.BlockSpec((_R, 1), lambda i: (i, 0)),
        ],
        out_specs=[
            pl.BlockSpec((_R, D), lambda i: (i, 0)),
            pl.BlockSpec((_R, 1), lambda i: (i, 0)),
        ],
        out_shape=[
            jax.ShapeDtypeStruct((N, D), jnp.float32),
            jax.ShapeDtypeStruct((N, 1), jnp.float32),
        ],
    )(t1, d0, d1)


def _tc_layer2(a0, a1, y1, dinv, b1, W2):
    def body(a0_ref, a1_ref, y1_ref, dinv_ref, b_ref, w_ref, y2_ref):
        dinv = dinv_ref[...]
        h = jnp.maximum(
            dinv * (a0_ref[...] + a1_ref[...] + y1_ref[...]) + b_ref[...],
            0.0)
        y2_ref[...] = jnp.dot(h, w_ref[...],
                              preferred_element_type=jnp.float32) * dinv

    return pl.pallas_call(
        body,
        grid=(N // _R,),
        in_specs=[
            pl.BlockSpec((_R, D), lambda i: (i, 0)),
            pl.BlockSpec((_R, D), lambda i: (i, 0)),
            pl.BlockSpec((_R, D), lambda i: (i, 0)),
            pl.BlockSpec((_R, 1), lambda i: (i, 0)),
            pl.BlockSpec((1, D), lambda i: (0, 0)),
            pl.BlockSpec((D, D), lambda i: (0, 0)),
        ],
        out_specs=pl.BlockSpec((_R, D), lambda i: (i, 0)),
        out_shape=jax.ShapeDtypeStruct((N, D), jnp.float32),
    )(a0, a1, y1, dinv, b1, W2)


def _tc_final(a0, a1, y2, dinv, b2):
    def body(a0_ref, a1_ref, y2_ref, dinv_ref, b_ref, out_ref):
        out_ref[...] = jnp.maximum(
            dinv_ref[...] * (a0_ref[...] + a1_ref[...] + y2_ref[...])
            + b_ref[...], 0.0)

    return pl.pallas_call(
        body,
        grid=(N // _R,),
        in_specs=[
            pl.BlockSpec((_R, D), lambda i: (i, 0)),
            pl.BlockSpec((_R, D), lambda i: (i, 0)),
            pl.BlockSpec((_R, D), lambda i: (i, 0)),
            pl.BlockSpec((_R, 1), lambda i: (i, 0)),
            pl.BlockSpec((1, D), lambda i: (0, 0)),
        ],
        out_specs=pl.BlockSpec((_R, D), lambda i: (i, 0)),
        out_shape=jax.ShapeDtypeStruct((N, D), jnp.float32),
    )(a0, a1, y2, dinv, b2)


_PAD_SRC = np.asarray((np.arange(PADE) * 13) % N, np.int32)
_PAD_DST = np.asarray(N + np.arange(PADE) % (NR - N), np.int32)


def kernel(x, edge_index, W1, b1, W2, b2):
    src = edge_index[0].astype(jnp.int32)
    dst = edge_index[1].astype(jnp.int32)

    srcp = jnp.concatenate([src, jnp.asarray(_PAD_SRC)]).reshape(NW, CH, B)
    dstp = jnp.concatenate([dst, jnp.asarray(_PAD_DST)]).reshape(NW, CH, B)

    onesrows = jnp.ones((B, D), jnp.float32)
    zrows = jnp.zeros((NR, D), jnp.float32)

    degp = _sc_degree(dstp, zrows, onesrows)
    d0 = degp[0, :N, 0:1]
    d1 = degp[1, :N, 0:1]

    y1, dinv = _tc_scale(_tc_matmul(x, W1), d0, d1)

    acc1 = _sc_scatter(y1, srcp, dstp, zrows)
    y2 = _tc_layer2(acc1[0, :N], acc1[1, :N], y1, dinv,
                    b1.reshape(1, D), W2)

    acc2 = _sc_scatter(y2, srcp, dstp, zrows)
    return _tc_final(acc2[0, :N], acc2[1, :N], y2, dinv, b2.reshape(1, D))


# speedup vs baseline: 1.1076x; 1.0025x over previous
import functools

import jax
import jax.numpy as jnp
import numpy as np
from jax import lax
from jax.experimental import pallas as pl
from jax.experimental.pallas import tpu as pltpu
from jax.experimental.pallas import tpu_sc as plsc

N = 10000
D = 128
E = 320000
NC = 2
NS = 16
NW = NC * NS
B = 128
CH = 80
EPW = CH * B
EP = NW * EPW
PADE = EP - E
NR = 10240
STRIPE = NR // NS

_mesh = plsc.VectorSubcoreMesh(core_axis_name="c", subcore_axis_name="s")


def _sc_degree(dstp, zrows, onesrows):
    @functools.partial(
        pl.kernel,
        out_type=jax.ShapeDtypeStruct((NC, NR, D), jnp.float32),
        mesh=_mesh,
        scratch_types=[
            pltpu.VMEM((CH, B), jnp.int32),
            pltpu.VMEM((B, D), jnp.float32),
            pltpu.VMEM_SHARED((NR, D), jnp.float32),
            pltpu.SemaphoreType.DMA,
        ],
    )
    def k(dstp_hbm, zeros_hbm, ones_hbm, out_hbm, idx_v, ones_v, degw, sem):
        cid = lax.axis_index("c")
        sid = lax.axis_index("s")
        wid = cid * NS + sid
        row0 = sid * STRIPE
        pltpu.sync_copy(zeros_hbm.at[pl.ds(row0, STRIPE)],
                        degw.at[pl.ds(row0, STRIPE)])
        pltpu.sync_copy(dstp_hbm.at[wid], idx_v)
        pltpu.sync_copy(ones_hbm, ones_v)
        plsc.subcore_barrier()

        K = 8

        def body(g, carry):
            for j in range(K):
                pltpu.async_copy(ones_v, degw.at[idx_v.at[g * K + j]],
                                 sem, add=True)
            for j in range(K):
                pltpu.make_async_copy(ones_v,
                                      degw.at[idx_v.at[g * K + j]],
                                      sem).wait()
            return carry

        lax.fori_loop(0, CH // K, body, 0)
        plsc.subcore_barrier()
        pltpu.sync_copy(degw.at[pl.ds(row0, STRIPE)],
                        out_hbm.at[cid, pl.ds(row0, STRIPE)])

    return k(dstp, zrows, onesrows)


def _sc_scatter(y, srcp, dstp, zrows):
    @functools.partial(
        pl.kernel,
        out_type=jax.ShapeDtypeStruct((NC, NR, D), jnp.float32),
        mesh=_mesh,
        scratch_types=[
            pltpu.VMEM((CH // 2, B), jnp.int32),
            pltpu.VMEM((CH // 2, B), jnp.int32),
            pltpu.VMEM((B, D), jnp.float32),
            pltpu.VMEM((B, D), jnp.float32),
            pltpu.VMEM_SHARED((NR, D), jnp.float32),
            pltpu.SemaphoreType.DMA,
            pltpu.SemaphoreType.DMA,
        ],
    )
    def k(y_hbm, srcp_hbm, dstp_hbm, z_hbm, out_hbm,
          sidx, didx, rows_a, rows_b, acc, sem_a, sem_b):
        cid = lax.axis_index("c")
        sid = lax.axis_index("s")
        wid = cid * NS + sid
        row0 = sid * STRIPE
        CH2 = CH // 2
        pltpu.sync_copy(z_hbm.at[pl.ds(row0, STRIPE)],
                        acc.at[pl.ds(row0, STRIPE)])
        plsc.subcore_barrier()

        for h in range(2):
            pltpu.sync_copy(srcp_hbm.at[wid, pl.ds(h * CH2, CH2)], sidx)
            pltpu.sync_copy(dstp_hbm.at[wid, pl.ds(h * CH2, CH2)], didx)
            pltpu.async_copy(y_hbm.at[sidx.at[0]], rows_a, sem_a)
            pltpu.async_copy(y_hbm.at[sidx.at[1]], rows_b, sem_b)

            def body(g, carry):
                ca = 2 * g
                pltpu.make_async_copy(y_hbm.at[sidx.at[ca]], rows_a,
                                      sem_a).wait()
                pltpu.sync_copy(rows_a, acc.at[didx.at[ca]], add=True)
                nxa = jnp.minimum(ca + 2, CH2 - 1)
                pltpu.async_copy(y_hbm.at[sidx.at[nxa]], rows_a, sem_a)
                pltpu.make_async_copy(y_hbm.at[sidx.at[ca + 1]], rows_b,
                                      sem_b).wait()
                pltpu.sync_copy(rows_b, acc.at[didx.at[ca + 1]], add=True)
                nxb = jnp.minimum(ca + 3, CH2 - 1)
                pltpu.async_copy(y_hbm.at[sidx.at[nxb]], rows_b, sem_b)
                return carry

            lax.fori_loop(0, CH2 // 2, body, 0)
            pltpu.make_async_copy(y_hbm.at[sidx.at[CH2 - 1]], rows_a,
                                  sem_a).wait()
            pltpu.make_async_copy(y_hbm.at[sidx.at[CH2 - 1]], rows_b,
                                  sem_b).wait()
        plsc.subcore_barrier()
        pltpu.sync_copy(acc.at[pl.ds(row0, STRIPE)],
                        out_hbm.at[cid, pl.ds(row0, STRIPE)])

    return k(y, srcp, dstp, zrows)


_R = 2000


def _tc_matmul(x, W1):
    def body(x_ref, w_ref, t_ref):
        t_ref[...] = jnp.dot(x_ref[...], w_ref[...],
                             preferred_element_type=jnp.float32)

    return pl.pallas_call(
        body,
        grid=(N // _R,),
        in_specs=[
            pl.BlockSpec((_R, D), lambda i: (i, 0)),
            pl.BlockSpec((D, D), lambda i: (0, 0)),
        ],
        out_specs=pl.BlockSpec((_R, D), lambda i: (i, 0)),
        out_shape=jax.ShapeDtypeStruct((N, D), jnp.float32),
    )(x, W1)


def _tc_scale(t1, d0, d1):
    def body(t_ref, d0_ref, d1_ref, y_ref, dinv_ref):
        dinv = lax.rsqrt(1.0 + d0_ref[...] + d1_ref[...])
        y_ref[...] = t_ref[...] * dinv
        dinv_ref[...] = dinv

    return pl.pallas_call(
        body,
        grid=(N // _R,),
        in_specs=[
            pl.BlockSpec((_R, D), lambda i: (i, 0)),
            pl.BlockSpec((_R, 1), lambda i: (i, 0)),
            pl.BlockSpec((_R, 1), lambda i: (i, 0)),
        ],
        out_specs=[
            pl.BlockSpec((_R, D), lambda i: (i, 0)),
            pl.BlockSpec((_R, 1), lambda i: (i, 0)),
        ],
        out_shape=[
            jax.ShapeDtypeStruct((N, D), jnp.float32),
            jax.ShapeDtypeStruct((N, 1), jnp.float32),
        ],
    )(t1, d0, d1)


def _tc_layer2(a0, a1, y1, dinv, b1, W2):
    def body(a0_ref, a1_ref, y1_ref, dinv_ref, b_ref, w_ref, y2_ref):
        dinv = dinv_ref[...]
        h = jnp.maximum(
            dinv * (a0_ref[...] + a1_ref[...] + y1_ref[...]) + b_ref[...],
            0.0)
        y2_ref[...] = jnp.dot(h, w_ref[...],
                              preferred_element_type=jnp.float32) * dinv

    return pl.pallas_call(
        body,
        grid=(N // _R,),
        in_specs=[
            pl.BlockSpec((_R, D), lambda i: (i, 0)),
            pl.BlockSpec((_R, D), lambda i: (i, 0)),
            pl.BlockSpec((_R, D), lambda i: (i, 0)),
            pl.BlockSpec((_R, 1), lambda i: (i, 0)),
            pl.BlockSpec((1, D), lambda i: (0, 0)),
            pl.BlockSpec((D, D), lambda i: (0, 0)),
        ],
        out_specs=pl.BlockSpec((_R, D), lambda i: (i, 0)),
        out_shape=jax.ShapeDtypeStruct((N, D), jnp.float32),
    )(a0, a1, y1, dinv, b1, W2)


def _tc_final(a0, a1, y2, dinv, b2):
    def body(a0_ref, a1_ref, y2_ref, dinv_ref, b_ref, out_ref):
        out_ref[...] = jnp.maximum(
            dinv_ref[...] * (a0_ref[...] + a1_ref[...] + y2_ref[...])
            + b_ref[...], 0.0)

    return pl.pallas_call(
        body,
        grid=(N // _R,),
        in_specs=[
            pl.BlockSpec((_R, D), lambda i: (i, 0)),
            pl.BlockSpec((_R, D), lambda i: (i, 0)),
            pl.BlockSpec((_R, D), lambda i: (i, 0)),
            pl.BlockSpec((_R, 1), lambda i: (i, 0)),
            pl.BlockSpec((1, D), lambda i: (0, 0)),
        ],
        out_specs=pl.BlockSpec((_R, D), lambda i: (i, 0)),
        out_shape=jax.ShapeDtypeStruct((N, D), jnp.float32),
    )(a0, a1, y2, dinv, b2)


_PAD_SRC = np.asarray((np.arange(PADE) * 13) % N, np.int32)
_PAD_DST = np.asarray(N + np.arange(PADE) % (NR - N), np.int32)


def kernel(x, edge_index, W1, b1, W2, b2):
    src = edge_index[0].astype(jnp.int32)
    dst = edge_index[1].astype(jnp.int32)

    srcp = jnp.concatenate([src, jnp.asarray(_PAD_SRC)]).reshape(NW, CH, B)
    dstp = jnp.concatenate([dst, jnp.asarray(_PAD_DST)]).reshape(NW, CH, B)

    onesrows = jnp.ones((B, D), jnp.float32)
    zrows = jnp.zeros((NR, D), jnp.float32)

    degp = _sc_degree(dstp, zrows, onesrows)
    d0 = degp[0, :N, 0:1]
    d1 = degp[1, :N, 0:1]

    y1, dinv = _tc_scale(_tc_matmul(x, W1), d0, d1)

    acc1 = _sc_scatter(y1, srcp, dstp, zrows)
    y2 = _tc_layer2(acc1[0, :N], acc1[1, :N], y1, dinv,
                    b1.reshape(1, D), W2)

    acc2 = _sc_scatter(y2, srcp, dstp, zrows)
    return _tc_final(acc2[0, :N], acc2[1, :N], y2, dinv, b2.reshape(1, D))


# speedup vs baseline: 1.1603x; 1.0476x over previous
import functools

import jax
import jax.numpy as jnp
import numpy as np
from jax import lax
from jax.experimental import pallas as pl
from jax.experimental.pallas import tpu as pltpu
from jax.experimental.pallas import tpu_sc as plsc

N = 10000
D = 128
E = 320000
NC = 2
NS = 16
NW = NC * NS
B = 128
CH = 80
EPW = CH * B
EP = NW * EPW
PADE = EP - E
NR = 10240
STRIPE = NR // NS

_mesh = plsc.VectorSubcoreMesh(core_axis_name="c", subcore_axis_name="s")


_NCH = E // B
_HI = 24
_TAIL0 = 2496


def _sc_degree(dst2d, zrows, onesrows):
    @functools.partial(
        pl.kernel,
        out_type=jax.ShapeDtypeStruct((NC, NR, D), jnp.float32),
        mesh=_mesh,
        scratch_types=[
            pltpu.VMEM((80, B), jnp.int32),
            pltpu.VMEM((4, B), jnp.int32),
            pltpu.VMEM((B, D), jnp.float32),
            pltpu.VMEM_SHARED((NR, D), jnp.float32),
        ],
    )
    def k(dst2d_hbm, zeros_hbm, ones_hbm, out_hbm, idx_v, tail_v, ones_v,
          degw):
        cid = lax.axis_index("c")
        sid = lax.axis_index("s")
        wid = cid * NS + sid
        row0 = sid * STRIPE
        pltpu.sync_copy(zeros_hbm.at[pl.ds(row0, STRIPE)],
                        degw.at[pl.ds(row0, STRIPE)])
        nch = jnp.where(wid < _HI, 80, 72)
        base = jnp.where(wid < _HI, wid * 80,
                         _HI * 80 + (wid - _HI) * 72)
        off0 = jnp.where(wid < _HI, 0, 8)
        pltpu.sync_copy(dst2d_hbm.at[pl.ds(base - off0, 80)], idx_v)
        pltpu.sync_copy(dst2d_hbm.at[pl.ds(_TAIL0, 4)], tail_v)
        pltpu.sync_copy(ones_hbm, ones_v)
        plsc.subcore_barrier()

        def body(ch, carry):
            pltpu.sync_copy(ones_v, degw.at[idx_v.at[off0 + ch]], add=True)
            return carry

        lax.fori_loop(0, nch, body, 0)

        def tbody(t, carry):
            pltpu.sync_copy(ones_v, degw.at[tail_v.at[wid]], add=True)
            return carry

        lax.fori_loop(0, jnp.where(wid < 4, 1, 0), tbody, 0)
        plsc.subcore_barrier()
        pltpu.sync_copy(degw.at[pl.ds(row0, STRIPE)],
                        out_hbm.at[cid, pl.ds(row0, STRIPE)])

    return k(dst2d, zrows, onesrows)


def _sc_scatter(y, srcp, dstp, zrows):
    @functools.partial(
        pl.kernel,
        out_type=jax.ShapeDtypeStruct((NC, NR, D), jnp.float32),
        mesh=_mesh,
        scratch_types=[
            pltpu.VMEM((CH // 2, B), jnp.int32),
            pltpu.VMEM((CH // 2, B), jnp.int32),
            pltpu.VMEM((B, D), jnp.float32),
            pltpu.VMEM((B, D), jnp.float32),
            pltpu.VMEM_SHARED((NR, D), jnp.float32),
            pltpu.SemaphoreType.DMA,
            pltpu.SemaphoreType.DMA,
        ],
    )
    def k(y_hbm, srcp_hbm, dstp_hbm, z_hbm, out_hbm,
          sidx, didx, rows_a, rows_b, acc, sem_a, sem_b):
        cid = lax.axis_index("c")
        sid = lax.axis_index("s")
        wid = cid * NS + sid
        row0 = sid * STRIPE
        CH2 = CH // 2
        pltpu.sync_copy(z_hbm.at[pl.ds(row0, STRIPE)],
                        acc.at[pl.ds(row0, STRIPE)])
        plsc.subcore_barrier()

        for h in range(2):
            pltpu.sync_copy(srcp_hbm.at[wid, pl.ds(h * CH2, CH2)], sidx)
            pltpu.sync_copy(dstp_hbm.at[wid, pl.ds(h * CH2, CH2)], didx)
            pltpu.async_copy(y_hbm.at[sidx.at[0]], rows_a, sem_a)
            pltpu.async_copy(y_hbm.at[sidx.at[1]], rows_b, sem_b)

            def body(g, carry):
                ca = 2 * g
                pltpu.make_async_copy(y_hbm.at[sidx.at[ca]], rows_a,
                                      sem_a).wait()
                pltpu.sync_copy(rows_a, acc.at[didx.at[ca]], add=True)
                nxa = jnp.minimum(ca + 2, CH2 - 1)
                pltpu.async_copy(y_hbm.at[sidx.at[nxa]], rows_a, sem_a)
                pltpu.make_async_copy(y_hbm.at[sidx.at[ca + 1]], rows_b,
                                      sem_b).wait()
                pltpu.sync_copy(rows_b, acc.at[didx.at[ca + 1]], add=True)
                nxb = jnp.minimum(ca + 3, CH2 - 1)
                pltpu.async_copy(y_hbm.at[sidx.at[nxb]], rows_b, sem_b)
                return carry

            lax.fori_loop(0, CH2 // 2, body, 0)
            pltpu.make_async_copy(y_hbm.at[sidx.at[CH2 - 1]], rows_a,
                                  sem_a).wait()
            pltpu.make_async_copy(y_hbm.at[sidx.at[CH2 - 1]], rows_b,
                                  sem_b).wait()
        plsc.subcore_barrier()
        pltpu.sync_copy(acc.at[pl.ds(row0, STRIPE)],
                        out_hbm.at[cid, pl.ds(row0, STRIPE)])

    return k(y, srcp, dstp, zrows)


_R = 2000


def _tc_matmul(x, W1):
    def body(x_ref, w_ref, t_ref):
        t_ref[...] = jnp.dot(x_ref[...], w_ref[...],
                             preferred_element_type=jnp.float32)

    return pl.pallas_call(
        body,
        grid=(N // _R,),
        in_specs=[
            pl.BlockSpec((_R, D), lambda i: (i, 0)),
            pl.BlockSpec((D, D), lambda i: (0, 0)),
        ],
        out_specs=pl.BlockSpec((_R, D), lambda i: (i, 0)),
        out_shape=jax.ShapeDtypeStruct((N, D), jnp.float32),
    )(x, W1)


def _tc_scale(t1, degp):
    def body(t_ref, d0_ref, d1_ref, y_ref, dinv_ref):
        dinv = lax.rsqrt(1.0 + d0_ref[0, :, 0:1] + d1_ref[0, :, 0:1])
        y_ref[...] = t_ref[...] * dinv
        dinv_ref[...] = dinv

    return pl.pallas_call(
        body,
        grid=(N // _R,),
        in_specs=[
            pl.BlockSpec((_R, D), lambda i: (i, 0)),
            pl.BlockSpec((1, _R, D), lambda i: (0, i, 0)),
            pl.BlockSpec((1, _R, D), lambda i: (1, i, 0)),
        ],
        out_specs=[
            pl.BlockSpec((_R, D), lambda i: (i, 0)),
            pl.BlockSpec((_R, 1), lambda i: (i, 0)),
        ],
        out_shape=[
            jax.ShapeDtypeStruct((N, D), jnp.float32),
            jax.ShapeDtypeStruct((N, 1), jnp.float32),
        ],
    )(t1, degp, degp)


def _tc_layer2(acc, y1, dinv, b1, W2):
    def body(a0_ref, a1_ref, y1_ref, dinv_ref, b_ref, w_ref, y2_ref):
        dinv = dinv_ref[...]
        h = jnp.maximum(
            dinv * (a0_ref[0] + a1_ref[0] + y1_ref[...]) + b_ref[...],
            0.0)
        y2_ref[...] = jnp.dot(h, w_ref[...],
                              preferred_element_type=jnp.float32) * dinv

    return pl.pallas_call(
        body,
        grid=(N // _R,),
        in_specs=[
            pl.BlockSpec((1, _R, D), lambda i: (0, i, 0)),
            pl.BlockSpec((1, _R, D), lambda i: (1, i, 0)),
            pl.BlockSpec((_R, D), lambda i: (i, 0)),
            pl.BlockSpec((_R, 1), lambda i: (i, 0)),
            pl.BlockSpec((1, D), lambda i: (0, 0)),
            pl.BlockSpec((D, D), lambda i: (0, 0)),
        ],
        out_specs=pl.BlockSpec((_R, D), lambda i: (i, 0)),
        out_shape=jax.ShapeDtypeStruct((N, D), jnp.float32),
    )(acc, acc, y1, dinv, b1, W2)


def _tc_final(acc, y2, dinv, b2):
    def body(a0_ref, a1_ref, y2_ref, dinv_ref, b_ref, out_ref):
        out_ref[...] = jnp.maximum(
            dinv_ref[...] * (a0_ref[0] + a1_ref[0] + y2_ref[...])
            + b_ref[...], 0.0)

    return pl.pallas_call(
        body,
        grid=(N // _R,),
        in_specs=[
            pl.BlockSpec((1, _R, D), lambda i: (0, i, 0)),
            pl.BlockSpec((1, _R, D), lambda i: (1, i, 0)),
            pl.BlockSpec((_R, D), lambda i: (i, 0)),
            pl.BlockSpec((_R, 1), lambda i: (i, 0)),
            pl.BlockSpec((1, D), lambda i: (0, 0)),
        ],
        out_specs=pl.BlockSpec((_R, D), lambda i: (i, 0)),
        out_shape=jax.ShapeDtypeStruct((N, D), jnp.float32),
    )(acc, acc, y2, dinv, b2)


_PAD_SRC = np.asarray((np.arange(PADE) * 13) % N, np.int32)
_PAD_DST = np.asarray(N + np.arange(PADE) % (NR - N), np.int32)


def kernel(x, edge_index, W1, b1, W2, b2):
    src = edge_index[0].astype(jnp.int32)
    dst = edge_index[1].astype(jnp.int32)

    srcp = jnp.concatenate([src, jnp.asarray(_PAD_SRC)]).reshape(NW, CH, B)
    dstp = jnp.concatenate([dst, jnp.asarray(_PAD_DST)]).reshape(NW, CH, B)

    onesrows = jnp.ones((B, D), jnp.float32)
    zrows = jnp.zeros((NR, D), jnp.float32)

    degp = _sc_degree(dst.reshape(_NCH, B), zrows, onesrows)
    y1, dinv = _tc_scale(_tc_matmul(x, W1), degp)

    acc1 = _sc_scatter(y1, srcp, dstp, zrows)
    y2 = _tc_layer2(acc1, y1, dinv, b1.reshape(1, D), W2)

    acc2 = _sc_scatter(y2, srcp, dstp, zrows)
    return _tc_final(acc2, y2, dinv, b2.reshape(1, D))
